# baseline (speedup 1.0000x reference)
import functools

import jax
import jax.numpy as jnp
from jax import lax
from jax.experimental import pallas as pl
from jax.experimental.pallas import tpu as pltpu
from jax.experimental.pallas import tpu_sc as plsc

def _dot3(a, b):
    ah = a.astype(jnp.bfloat16)
    al = (a - ah.astype(jnp.float32)).astype(jnp.bfloat16)
    bh = b.astype(jnp.bfloat16)
    bl = (b - bh.astype(jnp.float32)).astype(jnp.bfloat16)
    d = jnp.dot(al, bh, preferred_element_type=jnp.float32)
    d = d + jnp.dot(ah, bl, preferred_element_type=jnp.float32)
    d = d + jnp.dot(ah, bh, preferred_element_type=jnp.float32)
    return d

F = 1262
FP = 1280
N_NODES = 5120
N_EDGES = 15360
N_GRAPHS = 64

NC = 2
NS = 16
NW = NC * NS
L = 16


def _pad2(w, r, c):
    return jnp.pad(w, ((0, r - w.shape[0]), (0, c - w.shape[1])))


def _pad1(b, n):
    return jnp.pad(b, (0, n - b.shape[0]))


def _tc_node_kernel(x_ref, w1_ref, b1_ref, wd_ref, ws_ref, wh_ref,
                    a_ref, b_ref, c_ref):
    h0 = jnp.maximum(
        _dot3(x_ref[...], w1_ref[...])
        + b1_ref[...], 0.0)
    a_ref[...] = _dot3(h0, wd_ref[...])
    b_ref[...] = _dot3(h0, ws_ref[...])
    c_ref[...] = _dot3(h0, wh_ref[...])


def _tc_node(x, w1, b1, wd, ws, wh, bn=256):
    nb = N_NODES // bn
    full = pl.BlockSpec((FP, FP), lambda i: (0, 0))
    row = pl.BlockSpec((1, FP), lambda i: (0, 0))
    blk = pl.BlockSpec((bn, FP), lambda i: (i, 0))
    return pl.pallas_call(
        _tc_node_kernel,
        grid=(nb,),
        in_specs=[blk, full, row, full, full, full],
        out_specs=[blk, blk, blk],
        out_shape=[jax.ShapeDtypeStruct((N_NODES, FP), jnp.float32)] * 3,
    )(x, w1, b1, wd, ws, wh)


def _tc_table_kernel(emb_ref, encw_ref, encb_ref, we_ref, b0_ref, t_ref):
    e = _dot3(emb_ref[...], encw_ref[...]) + encb_ref[...]
    t_ref[...] = _dot3(e, we_ref[...]) + b0_ref[...]


def _tc_table(emb, encw, encb, we, b0):
    return pl.pallas_call(
        _tc_table_kernel,
        out_shape=jax.ShapeDtypeStruct((32, FP), jnp.float32),
    )(emb, encw, encb, we, b0)


_G_CH = 32


def _sc_gather_body(a_hbm, b_hbm, dst_hbm, src_hbm, out_hbm,
                    dbuf, sbuf, bufa, bufb, sem1, sem2):
    wid = lax.axis_index("s") * NC + lax.axis_index("c")
    epw = N_EDGES // NW
    base = wid * epw
    nch = epw // _G_CH

    def chunk(i, _):
        off = base + i * _G_CH
        pltpu.sync_copy(dst_hbm.at[pl.ds(off, _G_CH)], dbuf)
        pltpu.sync_copy(src_hbm.at[pl.ds(off, _G_CH)], sbuf)
        cpa = pltpu.async_copy(a_hbm.at[dbuf], bufa, sem1)
        cpb = pltpu.async_copy(b_hbm.at[sbuf], bufb, sem2)
        cpa.wait()
        cpb.wait()

        def row(j, _):
            for k in range(FP // L):
                sl = pl.ds(k * L, L)
                bufa[j, sl] = bufa[j, sl] + bufb[j, sl]
            return 0

        lax.fori_loop(0, _G_CH, row, 0)
        pltpu.sync_copy(bufa, out_hbm.at[pl.ds(off, _G_CH)])
        return 0

    lax.fori_loop(0, nch, chunk, 0)


def _sc_gather(a, b, dst, src):
    mesh = plsc.VectorSubcoreMesh(core_axis_name="c", subcore_axis_name="s",
                                  num_cores=NC, num_subcores=NS)
    fn = functools.partial(
        pl.kernel,
        out_type=jax.ShapeDtypeStruct((N_EDGES, FP), jnp.float32),
        mesh=mesh,
        compiler_params=pltpu.CompilerParams(needs_layout_passes=False),
        scratch_types=[
            pltpu.VMEM((_G_CH,), jnp.int32),
            pltpu.VMEM((_G_CH,), jnp.int32),
            pltpu.VMEM((_G_CH, FP), jnp.float32),
            pltpu.VMEM((_G_CH, FP), jnp.float32),
            pltpu.SemaphoreType.DMA,
            pltpu.SemaphoreType.DMA,
        ],
    )(_sc_gather_body)
    return fn(a, b, dst, src)


def _tc_edge_kernel(m0_ref, attr_ref, t_ref, w1_ref, b1_ref, w2_ref, b2_ref,
                    w3_ref, b3_ref, w4_ref, b4_ref, out_ref):
    attr = attr_ref[...]
    onehot = (attr == lax.broadcasted_iota(jnp.int32, (1, 32), 1)
              ).astype(jnp.float32)
    z = m0_ref[...] + _dot3(onehot, t_ref[...])
    for w_ref, b_ref in ((w1_ref, b1_ref), (w2_ref, b2_ref),
                         (w3_ref, b3_ref), (w4_ref, b4_ref)):
        z = _dot3(jnp.maximum(z, 0.0), w_ref[...]) + b_ref[...]
    out_ref[...] = z


def _tc_edge(m0, attr, t, ws, bs, bn=512):
    nb = N_EDGES // bn
    full = pl.BlockSpec((FP, FP), lambda i: (0, 0))
    row = pl.BlockSpec((1, FP), lambda i: (0, 0))
    blk = pl.BlockSpec((bn, FP), lambda i: (i, 0))
    ins = [blk,
           pl.BlockSpec((bn, 1), lambda i: (i, 0)),
           pl.BlockSpec((32, FP), lambda i: (0, 0))]
    args = [m0, attr, t]
    for w, b in zip(ws, bs):
        ins += [full, row]
        args += [w, b]
    return pl.pallas_call(
        _tc_edge_kernel,
        grid=(nb,),
        in_specs=ins,
        out_specs=blk,
        out_shape=jax.ShapeDtypeStruct((N_EDGES, FP), jnp.float32),
    )(*args)


_R_NPW = N_NODES // NW
_R_FB = 128
_R_NFC = FP // _R_FB
_R_ELMAX = 2048
_R_CE = 128
_R_DCH = 1024

_NEG = -3.0e38
_POS = 3.0e38


def _sc_agg_body(m_hbm, dst_hbm, s_hbm, q_hbm, mn_hbm, mx_hbm, cnt_hbm,
                 dall, elist, dloc, rowid, rowbuf,
                 accs, accq, accmn, accmx, acccnt, sem):
    wid = lax.axis_index("s") * NC + lax.axis_index("c")
    lo = wid * _R_NPW

    def zinit(k, _):
        z = jnp.zeros((L,), jnp.int32)
        elist[pl.ds(k * L, L)] = z
        dloc[pl.ds(k * L, L)] = z
        return 0
    lax.fori_loop(0, _R_ELMAX // L, zinit, 0)

    def dchunk(h, cur0):
        pltpu.sync_copy(dst_hbm.at[pl.ds(h * _R_DCH, _R_DCH)], dall)

        def scan(k, cur):
            v = dall[pl.ds(k * L, L)]
            msk = (v >= lo) & (v < lo + _R_NPW)
            lane = lax.iota(jnp.int32, L)
            eid = h * _R_DCH + k * L + lane
            mi = msk.astype(jnp.int32)
            pos = cur + plsc.cumsum(mi) - mi
            idx = jnp.where(msk, pos, _R_ELMAX + lane)
            plsc.store_scatter(elist, [idx], eid)
            plsc.store_scatter(dloc, [idx], v - lo)
            return cur + jnp.sum(mi)
        return lax.fori_loop(0, _R_DCH // L, scan, cur0)
    nloc = lax.fori_loop(0, N_EDGES // _R_DCH, dchunk, 0)

    nchunks = (nloc + _R_CE - 1) // _R_CE

    def fchunk(c, _):
        def ainit(j, _):
            for k in range(_R_FB // L):
                sl = pl.ds(k * L, L)
                accs[j, sl] = jnp.zeros((L,), jnp.float32)
                accq[j, sl] = jnp.zeros((L,), jnp.float32)
                accmn[j, sl] = jnp.full((L,), _POS, jnp.float32)
                accmx[j, sl] = jnp.full((L,), _NEG, jnp.float32)
            return 0
        lax.fori_loop(0, _R_NPW, ainit, 0)

        @pl.when(c == 0)
        def _():
            def cinit(j, _):
                acccnt[j, :] = jnp.zeros((L,), jnp.float32)
                return 0
            lax.fori_loop(0, _R_NPW, cinit, 0)

        def echunk(ec, _):
            ebase = ec * _R_CE

            def ridx(k, _):
                e = elist[pl.ds(ebase + k * L, L)]
                rowid[pl.ds(k * L, L)] = e * _R_NFC + c
                return 0
            lax.fori_loop(0, _R_CE // L, ridx, 0)
            pltpu.async_copy(m_hbm.at[rowid], rowbuf, sem).wait()

            nhere = jnp.minimum(nloc - ebase, _R_CE)

            def edge(e, _):
                d = dloc[pl.ds(ebase + e, L)][0]
                for k in range(_R_FB // L):
                    sl = pl.ds(k * L, L)
                    v = rowbuf[e, sl]
                    accs[d, sl] = accs[d, sl] + v
                    accq[d, sl] = accq[d, sl] + v * v
                    accmn[d, sl] = jnp.minimum(accmn[d, sl], v)
                    accmx[d, sl] = jnp.maximum(accmx[d, sl], v)

                @pl.when(c == 0)
                def _():
                    acccnt[d, :] = acccnt[d, :] + 1.0
                return 0
            lax.fori_loop(0, nhere, edge, 0)
            return 0
        lax.fori_loop(0, nchunks, echunk, 0)

        fsl = pl.ds(c * _R_FB, _R_FB)
        nsl = pl.ds(lo, _R_NPW)
        pltpu.sync_copy(accs, s_hbm.at[nsl, fsl])
        pltpu.sync_copy(accq, q_hbm.at[nsl, fsl])
        pltpu.sync_copy(accmn, mn_hbm.at[nsl, fsl])
        pltpu.sync_copy(accmx, mx_hbm.at[nsl, fsl])

        @pl.when(c == 0)
        def _():
            pltpu.sync_copy(acccnt, cnt_hbm.at[nsl])
        return 0

    lax.fori_loop(0, _R_NFC, fchunk, 0)


def _sc_agg(m, dst):
    mesh = plsc.VectorSubcoreMesh(core_axis_name="c", subcore_axis_name="s",
                                  num_cores=NC, num_subcores=NS)
    mview = m.reshape(N_EDGES * _R_NFC, _R_FB)
    nf = jax.ShapeDtypeStruct((N_NODES, FP), jnp.float32)
    fn = functools.partial(
        pl.kernel,
        out_type=[nf, nf, nf, nf,
                  jax.ShapeDtypeStruct((N_NODES, L), jnp.float32)],
        mesh=mesh,
        compiler_params=pltpu.CompilerParams(needs_layout_passes=False),
        scratch_types=[
            pltpu.VMEM((_R_DCH,), jnp.int32),
            pltpu.VMEM((_R_ELMAX + L,), jnp.int32),
            pltpu.VMEM((_R_ELMAX + L,), jnp.int32),
            pltpu.VMEM((_R_CE,), jnp.int32),
            pltpu.VMEM((_R_CE, _R_FB), jnp.float32),
            pltpu.VMEM((_R_NPW, _R_FB), jnp.float32),
            pltpu.VMEM((_R_NPW, _R_FB), jnp.float32),
            pltpu.VMEM((_R_NPW, _R_FB), jnp.float32),
            pltpu.VMEM((_R_NPW, _R_FB), jnp.float32),
            pltpu.VMEM((_R_NPW, L), jnp.float32),
            pltpu.SemaphoreType.DMA,
        ],
    )(_sc_agg_body)
    return fn(mview, dst)


def _tc_post0_kernel(c_ref, s_ref, q_ref, mn_ref, mx_ref, cnt_ref,
                     p1_ref, p2_ref, p3_ref, p4_ref, p5_ref, b0_ref, out_ref):
    cnt = cnt_ref[...][:, 0:1]
    inv = 1.0 / jnp.maximum(cnt, 1.0)
    has = (cnt > 0.0).astype(jnp.float32)
    s = s_ref[...]
    mean = s * inv
    msq = q_ref[...] * inv
    std = jnp.sqrt(jnp.maximum(msq - mean * mean, 0.0) + 1e-5)
    mn = mn_ref[...] * has
    mx = mx_ref[...] * has
    acc = c_ref[...] + b0_ref[...]
    acc += _dot3(s, p1_ref[...])
    acc += _dot3(mean, p2_ref[...])
    acc += _dot3(mn, p3_ref[...])
    acc += _dot3(mx, p4_ref[...])
    acc += _dot3(std, p5_ref[...])
    out_ref[...] = acc


def _tc_post0(c, s, q, mn, mx, cnt, p1, p2, p3, p4, p5, b0, bn=128):
    nb = N_NODES // bn
    full = pl.BlockSpec((FP, FP), lambda i: (0, 0))
    row = pl.BlockSpec((1, FP), lambda i: (0, 0))
    blk = pl.BlockSpec((bn, FP), lambda i: (i, 0))
    cblk = pl.BlockSpec((bn, L), lambda i: (i, 0))
    return pl.pallas_call(
        _tc_post0_kernel,
        grid=(nb,),
        in_specs=[blk, blk, blk, blk, blk, cblk,
                  full, full, full, full, full, row],
        out_specs=blk,
        out_shape=jax.ShapeDtypeStruct((N_NODES, FP), jnp.float32),
    )(c, s, q, mn, mx, cnt, p1, p2, p3, p4, p5, b0)


def _tc_post_chain_kernel(z_ref, w1_ref, b1_ref, w2_ref, b2_ref, w3_ref,
                          b3_ref, w4_ref, b4_ref, wl_ref, bl_ref,
                          out_ref, stat_ref):
    z = z_ref[...]
    for w_ref, b_ref in ((w1_ref, b1_ref), (w2_ref, b2_ref),
                         (w3_ref, b3_ref), (w4_ref, b4_ref)):
        z = _dot3(jnp.maximum(z, 0.0), w_ref[...]) + b_ref[...]
    z = _dot3(z, wl_ref[...]) + bl_ref[...]
    out_ref[...] = z

    @pl.when(pl.program_id(0) == 0)
    def _():
        stat_ref[...] = jnp.zeros_like(stat_ref)
    stat_ref[0:1, :] += jnp.sum(z, axis=0, keepdims=True)
    stat_ref[1:2, :] += jnp.sum(z * z, axis=0, keepdims=True)


def _tc_post_chain(z, ws, bs, bn=256):
    nb = N_NODES // bn
    full = pl.BlockSpec((FP, FP), lambda i: (0, 0))
    row = pl.BlockSpec((1, FP), lambda i: (0, 0))
    blk = pl.BlockSpec((bn, FP), lambda i: (i, 0))
    ins = [blk]
    args = [z]
    for w, b in zip(ws, bs):
        ins += [full, row]
        args += [w, b]
    return pl.pallas_call(
        _tc_post_chain_kernel,
        grid=(nb,),
        in_specs=ins,
        out_specs=[blk, pl.BlockSpec((8, FP), lambda i: (0, 0))],
        out_shape=[jax.ShapeDtypeStruct((N_NODES, FP), jnp.float32),
                   jax.ShapeDtypeStruct((8, FP), jnp.float32)],
    )(*args)


def _tc_head_kernel(z_ref, stat_ref, g_ref, be_ref, batch_ref,
                    w1_ref, b1_ref, w2_ref, b2_ref, w3_ref, b3_ref,
                    xf_ref, pool_ref):
    stat = stat_ref[...]
    mu = stat[0:1, :] * (1.0 / N_NODES)
    ex2 = stat[1:2, :] * (1.0 / N_NODES)
    var = ex2 - mu * mu
    rstd = jax.lax.rsqrt(var + 1e-5)
    z = (z_ref[...] - mu) * rstd * g_ref[...] + be_ref[...]
    h = jnp.maximum(z, 0.0)

    f = jnp.maximum(_dot3(h, w1_ref[...]) + b1_ref[...],
                    0.0)
    f = jnp.maximum(_dot3(f, w2_ref[...]) + b2_ref[...],
                    0.0)
    xf_ref[...] = _dot3(f, w3_ref[...]) + b3_ref[...]

    onehot = (batch_ref[...] == lax.broadcasted_iota(
        jnp.int32, (1, N_GRAPHS), 1)).astype(jnp.float32)

    @pl.when(pl.program_id(0) == 0)
    def _():
        pool_ref[...] = jnp.zeros_like(pool_ref)
    pool_ref[...] += _dot3(onehot.T, h)


def _tc_head(z, stat, g, be, batch, w1, b1, w2, b2, w3, b3, bn=256):
    nb = N_NODES // bn
    blk = pl.BlockSpec((bn, FP), lambda i: (i, 0))
    return pl.pallas_call(
        _tc_head_kernel,
        grid=(nb,),
        in_specs=[blk,
                  pl.BlockSpec((8, FP), lambda i: (0, 0)),
                  pl.BlockSpec((1, FP), lambda i: (0, 0)),
                  pl.BlockSpec((1, FP), lambda i: (0, 0)),
                  pl.BlockSpec((bn, 1), lambda i: (i, 0)),
                  pl.BlockSpec((FP, 640), lambda i: (0, 0)),
                  pl.BlockSpec((1, 640), lambda i: (0, 0)),
                  pl.BlockSpec((640, 128), lambda i: (0, 0)),
                  pl.BlockSpec((1, 128), lambda i: (0, 0)),
                  pl.BlockSpec((128, 128), lambda i: (0, 0)),
                  pl.BlockSpec((1, 128), lambda i: (0, 0))],
        out_specs=[pl.BlockSpec((bn, 128), lambda i: (i, 0)),
                   pl.BlockSpec((N_GRAPHS, FP), lambda i: (0, 0))],
        out_shape=[jax.ShapeDtypeStruct((N_NODES, 128), jnp.float32),
                   jax.ShapeDtypeStruct((N_GRAPHS, FP), jnp.float32)],
    )(z, stat, g, be, batch, w1, b1, w2, b2, w3, b3)


def _tc_energy_kernel(p_ref, w1_ref, b1_ref, w2_ref, b2_ref, w3_ref, b3_ref,
                      out_ref):
    f = jnp.maximum(_dot3(p_ref[...], w1_ref[...]) + b1_ref[...],
                    0.0)
    f = jnp.maximum(_dot3(f, w2_ref[...]) + b2_ref[...],
                    0.0)
    out_ref[...] = _dot3(f, w3_ref[...]) + b3_ref[...]


def _tc_energy(p, w1, b1, w2, b2, w3, b3):
    return pl.pallas_call(
        _tc_energy_kernel,
        out_shape=jax.ShapeDtypeStruct((N_GRAPHS, 128), jnp.float32),
    )(p, w1, b1, w2, b2, w3, b3)


def kernel(x, edge_index, edge_attr, batch, params):
    xp = jnp.pad(x, ((0, 0), (0, FP - F)))
    src = edge_index[0]
    dst = edge_index[1]

    w1 = _pad2(params["mlp1"]["w"], FP, FP)
    b1 = _pad1(params["mlp1"]["b"], FP).reshape(1, FP)

    pre0w = params["pre"][0]["w"]
    wd = _pad2(pre0w[:F], FP, FP)
    ws = _pad2(pre0w[F:2 * F], FP, FP)
    we = _pad2(pre0w[2 * F:], FP, FP)
    b0pre = _pad1(params["pre"][0]["b"], FP).reshape(1, FP)

    emb = _pad2(params["edge_emb"], 32, 16)
    encw = _pad2(params["edge_enc"]["w"], 16, FP)
    encb = _pad1(params["edge_enc"]["b"], FP).reshape(1, FP)

    prew = [_pad2(p["w"], FP, FP) for p in params["pre"][1:]]
    preb = [_pad1(p["b"], FP).reshape(1, FP) for p in params["pre"][1:]]

    aw = jax.nn.softmax(params["agg_w"])
    post0w = params["post"][0]["w"]
    wh = _pad2(post0w[:F], FP, FP)
    p1 = _pad2(post0w[F:2 * F], FP, FP) * aw[0]
    p2 = _pad2(post0w[2 * F:3 * F], FP, FP) * aw[1]
    p3 = _pad2(post0w[3 * F:4 * F], FP, FP) * aw[2]
    p4 = _pad2(post0w[4 * F:5 * F], FP, FP) * aw[3]
    p5 = _pad2(post0w[5 * F:], FP, FP) * aw[4]
    b0post = _pad1(params["post"][0]["b"], FP).reshape(1, FP)

    postw = [_pad2(p["w"], FP, FP) for p in params["post"][1:]]
    postb = [_pad1(p["b"], FP).reshape(1, FP) for p in params["post"][1:]]
    postw.append(_pad2(params["lin"]["w"], FP, FP))
    postb.append(_pad1(params["lin"]["b"], FP).reshape(1, FP))

    g = _pad1(params["bn_gamma"], FP).reshape(1, FP)
    be = _pad1(params["bn_beta"], FP).reshape(1, FP)

    m3 = params["mlp3"]
    m3w1 = _pad2(m3[0]["w"], FP, 640)
    m3b1 = _pad1(m3[0]["b"], 640).reshape(1, 640)
    m3w2 = _pad2(m3[1]["w"], 640, 128)
    m3b2 = _pad1(m3[1]["b"], 128).reshape(1, 128)
    m3w3 = _pad2(m3[2]["w"], 128, 128)
    m3b3 = _pad1(m3[2]["b"], 128).reshape(1, 128)

    m2 = params["mlp2"]
    m2w1 = _pad2(m2[0]["w"], FP, 640)
    m2b1 = _pad1(m2[0]["b"], 640).reshape(1, 640)
    m2w2 = _pad2(m2[1]["w"], 640, 128)
    m2b2 = _pad1(m2[1]["b"], 128).reshape(1, 128)
    m2w3 = _pad2(m2[2]["w"], 128, 128)
    m2b3 = _pad1(m2[2]["b"], 128).reshape(1, 128)

    a, b, c = _tc_node(xp, w1, b1, wd, ws, wh)
    t = _tc_table(emb, encw, encb, we, b0pre)
    m0 = _sc_gather(a, b, dst, src)
    m = _tc_edge(m0, edge_attr.astype(jnp.int32).reshape(N_EDGES, 1), t,
                 prew, preb)
    s, q, mn, mx, cnt = _sc_agg(m, dst)
    z0 = _tc_post0(c, s, q, mn, mx, cnt, p1, p2, p3, p4, p5, b0post)
    z, stat = _tc_post_chain(z0, postw, postb)
    xf, pool = _tc_head(z, stat, g, be, batch.astype(jnp.int32).reshape(
        N_NODES, 1), m3w1, m3b1, m3w2, m3b2, m3w3, m3b3)
    xe = _tc_energy(pool, m2w1, m2b1, m2w2, m2b2, m2w3, m2b3)

    force = xf[:, :3]
    energy = xe[:, :1]
    return force, energy


# speedup vs baseline: 1.0260x; 1.0260x over previous
import functools

import jax
import jax.numpy as jnp
from jax import lax
from jax.experimental import pallas as pl
from jax.experimental.pallas import tpu as pltpu
from jax.experimental.pallas import tpu_sc as plsc

def _split(a):
    ah = a.astype(jnp.bfloat16)
    al = (a - ah.astype(jnp.float32)).astype(jnp.bfloat16)
    return ah, al


def _mm(a2, b2):
    ah, al = a2
    bh, bl = b2
    d = jnp.dot(al, bh, preferred_element_type=jnp.float32)
    d = d + jnp.dot(ah, bl, preferred_element_type=jnp.float32)
    d = d + jnp.dot(ah, bh, preferred_element_type=jnp.float32)
    return d


def _dot3(a, b):
    return _mm(_split(a), _split(b))


_VMEM_BIG = pltpu.CompilerParams(vmem_limit_bytes=100 * 1024 * 1024)

F = 1262
FP = 1280
N_NODES = 5120
N_EDGES = 15360
N_GRAPHS = 64

NC = 2
NS = 16
NW = NC * NS
L = 16


def _pad2(w, r, c):
    return jnp.pad(w, ((0, r - w.shape[0]), (0, c - w.shape[1])))


def _pad1(b, n):
    return jnp.pad(b, (0, n - b.shape[0]))


def _wsplit(w):
    hi = w.astype(jnp.bfloat16)
    lo = (w - hi.astype(jnp.float32)).astype(jnp.bfloat16)
    return jnp.stack([hi, lo])


def _tc_node_kernel(x_ref, w1_ref, b1_ref, wd_ref, ws_ref, wh_ref,
                    a_ref, b_ref, c_ref):
    h0 = jnp.maximum(
        _mm(_split(x_ref[...]), (w1_ref[0], w1_ref[1]))
        + b1_ref[...], 0.0)
    h2 = _split(h0)
    a_ref[...] = _mm(h2, (wd_ref[0], wd_ref[1]))
    b_ref[...] = _mm(h2, (ws_ref[0], ws_ref[1]))
    c_ref[...] = _mm(h2, (wh_ref[0], wh_ref[1]))


def _tc_node(x, w1, b1, wd, ws, wh, bn=512):
    nb = N_NODES // bn
    full = pl.BlockSpec((2, FP, FP), lambda i: (0, 0, 0))
    row = pl.BlockSpec((1, FP), lambda i: (0, 0))
    blk = pl.BlockSpec((bn, FP), lambda i: (i, 0))
    return pl.pallas_call(
        _tc_node_kernel,
        grid=(nb,),
        in_specs=[blk, full, row, full, full, full],
        out_specs=[blk, blk, blk],
        out_shape=[jax.ShapeDtypeStruct((N_NODES, FP), jnp.float32)] * 3,
        compiler_params=_VMEM_BIG,
    )(x, w1, b1, wd, ws, wh)


def _tc_table_kernel(emb_ref, encw_ref, encb_ref, we_ref, b0_ref, t_ref):
    e = _dot3(emb_ref[...], encw_ref[...]) + encb_ref[...]
    t_ref[...] = _dot3(e, we_ref[...]) + b0_ref[...]


def _tc_table(emb, encw, encb, we, b0):
    return pl.pallas_call(
        _tc_table_kernel,
        out_shape=jax.ShapeDtypeStruct((32, FP), jnp.float32),
    )(emb, encw, encb, we, b0)


_G_CH = 32


def _sc_gather_body(a_hbm, b_hbm, dst_hbm, src_hbm, out_hbm,
                    dbuf, sbuf, bufa, bufb, sem1, sem2):
    wid = lax.axis_index("s") * NC + lax.axis_index("c")
    epw = N_EDGES // NW
    base = wid * epw
    nch = epw // _G_CH

    def chunk(i, _):
        off = base + i * _G_CH
        pltpu.sync_copy(dst_hbm.at[pl.ds(off, _G_CH)], dbuf)
        pltpu.sync_copy(src_hbm.at[pl.ds(off, _G_CH)], sbuf)
        cpa = pltpu.async_copy(a_hbm.at[dbuf], bufa, sem1)
        cpb = pltpu.async_copy(b_hbm.at[sbuf], bufb, sem2)
        cpa.wait()
        cpb.wait()

        def row(j, _):
            for k in range(FP // L):
                sl = pl.ds(k * L, L)
                bufa[j, sl] = bufa[j, sl] + bufb[j, sl]
            return 0

        lax.fori_loop(0, _G_CH, row, 0)
        pltpu.sync_copy(bufa, out_hbm.at[pl.ds(off, _G_CH)])
        return 0

    lax.fori_loop(0, nch, chunk, 0)


def _sc_gather(a, b, dst, src):
    mesh = plsc.VectorSubcoreMesh(core_axis_name="c", subcore_axis_name="s",
                                  num_cores=NC, num_subcores=NS)
    fn = functools.partial(
        pl.kernel,
        out_type=jax.ShapeDtypeStruct((N_EDGES, FP), jnp.float32),
        mesh=mesh,
        compiler_params=pltpu.CompilerParams(needs_layout_passes=False),
        scratch_types=[
            pltpu.VMEM((_G_CH,), jnp.int32),
            pltpu.VMEM((_G_CH,), jnp.int32),
            pltpu.VMEM((_G_CH, FP), jnp.float32),
            pltpu.VMEM((_G_CH, FP), jnp.float32),
            pltpu.SemaphoreType.DMA,
            pltpu.SemaphoreType.DMA,
        ],
    )(_sc_gather_body)
    return fn(a, b, dst, src)


def _tc_edge_kernel(m0_ref, attr_ref, t_ref, w1_ref, b1_ref, w2_ref, b2_ref,
                    w3_ref, b3_ref, w4_ref, b4_ref, out_ref):
    attr = attr_ref[...]
    onehot = (attr == lax.broadcasted_iota(jnp.int32, (1, 32), 1)
              ).astype(jnp.float32)
    z = m0_ref[...] + _dot3(onehot, t_ref[...])
    for w_ref, b_ref in ((w1_ref, b1_ref), (w2_ref, b2_ref),
                         (w3_ref, b3_ref), (w4_ref, b4_ref)):
        z = _mm(_split(jnp.maximum(z, 0.0)),
                (w_ref[0], w_ref[1])) + b_ref[...]
    out_ref[...] = z


def _tc_edge(m0, attr, t, ws, bs, bn=512):
    nb = N_EDGES // bn
    full = pl.BlockSpec((2, FP, FP), lambda i: (0, 0, 0))
    row = pl.BlockSpec((1, FP), lambda i: (0, 0))
    blk = pl.BlockSpec((bn, FP), lambda i: (i, 0))
    ins = [blk,
           pl.BlockSpec((bn, 1), lambda i: (i, 0)),
           pl.BlockSpec((32, FP), lambda i: (0, 0))]
    args = [m0, attr, t]
    for w, b in zip(ws, bs):
        ins += [full, row]
        args += [w, b]
    return pl.pallas_call(
        _tc_edge_kernel,
        grid=(nb,),
        in_specs=ins,
        out_specs=blk,
        out_shape=jax.ShapeDtypeStruct((N_EDGES, FP), jnp.float32),
        compiler_params=_VMEM_BIG,
    )(*args)


_R_NPW = N_NODES // NW
_R_FB = 128
_R_NFC = FP // _R_FB
_R_ELMAX = 2048
_R_CE = 128
_R_DCH = 1024

_NEG = -3.0e38
_POS = 3.0e38


def _sc_agg_body(m_hbm, dst_hbm, s_hbm, q_hbm, mn_hbm, mx_hbm, cnt_hbm,
                 dall, elist, dloc, rowid, rowbuf,
                 accs, accq, accmn, accmx, acccnt, sem):
    wid = lax.axis_index("s") * NC + lax.axis_index("c")
    lo = wid * _R_NPW

    def zinit(k, _):
        z = jnp.zeros((L,), jnp.int32)
        elist[pl.ds(k * L, L)] = z
        dloc[pl.ds(k * L, L)] = z
        return 0
    lax.fori_loop(0, _R_ELMAX // L, zinit, 0)

    def dchunk(h, cur0):
        pltpu.sync_copy(dst_hbm.at[pl.ds(h * _R_DCH, _R_DCH)], dall)

        def scan(k, cur):
            v = dall[pl.ds(k * L, L)]
            msk = (v >= lo) & (v < lo + _R_NPW)
            lane = lax.iota(jnp.int32, L)
            eid = h * _R_DCH + k * L + lane
            mi = msk.astype(jnp.int32)
            pos = cur + plsc.cumsum(mi) - mi
            idx = jnp.where(msk, pos, _R_ELMAX + lane)
            plsc.store_scatter(elist, [idx], eid)
            plsc.store_scatter(dloc, [idx], v - lo)
            return cur + jnp.sum(mi)
        return lax.fori_loop(0, _R_DCH // L, scan, cur0)
    nloc = lax.fori_loop(0, N_EDGES // _R_DCH, dchunk, 0)

    def cinit(j, _):
        acccnt[j, :] = jnp.zeros((L,), jnp.float32)
        return 0
    lax.fori_loop(0, _R_NPW, cinit, 0)

    def ccount(e, _):
        d = dloc[pl.ds(e, L)][0]
        acccnt[d, :] = acccnt[d, :] + 1.0
        return 0
    lax.fori_loop(0, nloc, ccount, 0)
    pltpu.sync_copy(acccnt, cnt_hbm.at[pl.ds(lo, _R_NPW)])

    nchunks = (nloc + _R_CE - 1) // _R_CE

    def fchunk(c, _):
        def ainit(j, _):
            for k in range(_R_FB // L):
                sl = pl.ds(k * L, L)
                accs[j, sl] = jnp.zeros((L,), jnp.float32)
                accq[j, sl] = jnp.zeros((L,), jnp.float32)
                accmn[j, sl] = jnp.full((L,), _POS, jnp.float32)
                accmx[j, sl] = jnp.full((L,), _NEG, jnp.float32)
            return 0
        lax.fori_loop(0, _R_NPW, ainit, 0)

        def echunk(ec, _):
            ebase = ec * _R_CE

            def ridx(k, _):
                e = elist[pl.ds(ebase + k * L, L)]
                rowid[pl.ds(k * L, L)] = e * _R_NFC + c
                return 0
            lax.fori_loop(0, _R_CE // L, ridx, 0)
            pltpu.async_copy(m_hbm.at[rowid], rowbuf, sem).wait()

            nhere = jnp.minimum(nloc - ebase, _R_CE)

            def edge(e, _):
                d = dloc[pl.ds(ebase + e, L)][0]
                nk = _R_FB // L
                sls = [pl.ds(k * L, L) for k in range(nk)]
                vs = [rowbuf[e, sl] for sl in sls]
                olds = [accs[d, sl] for sl in sls]
                oldq = [accq[d, sl] for sl in sls]
                oldmn = [accmn[d, sl] for sl in sls]
                oldmx = [accmx[d, sl] for sl in sls]
                news = [o + v for o, v in zip(olds, vs)]
                newq = [o + v * v for o, v in zip(oldq, vs)]
                newmn = [jnp.minimum(o, v) for o, v in zip(oldmn, vs)]
                newmx = [jnp.maximum(o, v) for o, v in zip(oldmx, vs)]
                for k, sl in enumerate(sls):
                    accs[d, sl] = news[k]
                    accq[d, sl] = newq[k]
                    accmn[d, sl] = newmn[k]
                    accmx[d, sl] = newmx[k]
                return 0
            lax.fori_loop(0, nhere, edge, 0)
            return 0
        lax.fori_loop(0, nchunks, echunk, 0)

        fsl = pl.ds(c * _R_FB, _R_FB)
        nsl = pl.ds(lo, _R_NPW)
        pltpu.sync_copy(accs, s_hbm.at[nsl, fsl])
        pltpu.sync_copy(accq, q_hbm.at[nsl, fsl])
        pltpu.sync_copy(accmn, mn_hbm.at[nsl, fsl])
        pltpu.sync_copy(accmx, mx_hbm.at[nsl, fsl])
        return 0

    lax.fori_loop(0, _R_NFC, fchunk, 0)


def _sc_agg(m, dst):
    mesh = plsc.VectorSubcoreMesh(core_axis_name="c", subcore_axis_name="s",
                                  num_cores=NC, num_subcores=NS)
    mview = m.reshape(N_EDGES * _R_NFC, _R_FB)
    nf = jax.ShapeDtypeStruct((N_NODES, FP), jnp.float32)
    fn = functools.partial(
        pl.kernel,
        out_type=[nf, nf, nf, nf,
                  jax.ShapeDtypeStruct((N_NODES, L), jnp.float32)],
        mesh=mesh,
        compiler_params=pltpu.CompilerParams(needs_layout_passes=False),
        scratch_types=[
            pltpu.VMEM((_R_DCH,), jnp.int32),
            pltpu.VMEM((_R_ELMAX + L,), jnp.int32),
            pltpu.VMEM((_R_ELMAX + L,), jnp.int32),
            pltpu.VMEM((_R_CE,), jnp.int32),
            pltpu.VMEM((_R_CE, _R_FB), jnp.float32),
            pltpu.VMEM((_R_NPW, _R_FB), jnp.float32),
            pltpu.VMEM((_R_NPW, _R_FB), jnp.float32),
            pltpu.VMEM((_R_NPW, _R_FB), jnp.float32),
            pltpu.VMEM((_R_NPW, _R_FB), jnp.float32),
            pltpu.VMEM((_R_NPW, L), jnp.float32),
            pltpu.SemaphoreType.DMA,
        ],
    )(_sc_agg_body)
    return fn(mview, dst)


def _tc_post0_kernel(c_ref, s_ref, q_ref, mn_ref, mx_ref, cnt_ref,
                     p1_ref, p2_ref, p3_ref, p4_ref, p5_ref, b0_ref, out_ref):
    cnt = cnt_ref[...][:, 0:1]
    inv = 1.0 / jnp.maximum(cnt, 1.0)
    has = (cnt > 0.0).astype(jnp.float32)
    s = s_ref[...]
    mean = s * inv
    msq = q_ref[...] * inv
    std = jnp.sqrt(jnp.maximum(msq - mean * mean, 0.0) + 1e-5)
    mn = mn_ref[...] * has
    mx = mx_ref[...] * has
    acc = c_ref[...] + b0_ref[...]
    acc += _mm(_split(s), (p1_ref[0], p1_ref[1]))
    acc += _mm(_split(mean), (p2_ref[0], p2_ref[1]))
    acc += _mm(_split(mn), (p3_ref[0], p3_ref[1]))
    acc += _mm(_split(mx), (p4_ref[0], p4_ref[1]))
    acc += _mm(_split(std), (p5_ref[0], p5_ref[1]))
    out_ref[...] = acc


def _tc_post0(c, s, q, mn, mx, cnt, p1, p2, p3, p4, p5, b0, bn=256):
    nb = N_NODES // bn
    full = pl.BlockSpec((2, FP, FP), lambda i: (0, 0, 0))
    row = pl.BlockSpec((1, FP), lambda i: (0, 0))
    blk = pl.BlockSpec((bn, FP), lambda i: (i, 0))
    cblk = pl.BlockSpec((bn, L), lambda i: (i, 0))
    return pl.pallas_call(
        _tc_post0_kernel,
        grid=(nb,),
        in_specs=[blk, blk, blk, blk, blk, cblk,
                  full, full, full, full, full, row],
        out_specs=blk,
        out_shape=jax.ShapeDtypeStruct((N_NODES, FP), jnp.float32),
        compiler_params=_VMEM_BIG,
    )(c, s, q, mn, mx, cnt, p1, p2, p3, p4, p5, b0)


def _tc_post_chain_kernel(z_ref, w1_ref, b1_ref, w2_ref, b2_ref, w3_ref,
                          b3_ref, w4_ref, b4_ref, wl_ref, bl_ref,
                          out_ref, stat_ref):
    z = z_ref[...]
    for w_ref, b_ref in ((w1_ref, b1_ref), (w2_ref, b2_ref),
                         (w3_ref, b3_ref), (w4_ref, b4_ref)):
        z = _mm(_split(jnp.maximum(z, 0.0)),
                (w_ref[0], w_ref[1])) + b_ref[...]
    z = _mm(_split(z), (wl_ref[0], wl_ref[1])) + bl_ref[...]
    out_ref[...] = z

    @pl.when(pl.program_id(0) == 0)
    def _():
        stat_ref[...] = jnp.zeros_like(stat_ref)
    stat_ref[0:1, :] += jnp.sum(z, axis=0, keepdims=True)
    stat_ref[1:2, :] += jnp.sum(z * z, axis=0, keepdims=True)


def _tc_post_chain(z, ws, bs, bn=256):
    nb = N_NODES // bn
    full = pl.BlockSpec((2, FP, FP), lambda i: (0, 0, 0))
    row = pl.BlockSpec((1, FP), lambda i: (0, 0))
    blk = pl.BlockSpec((bn, FP), lambda i: (i, 0))
    ins = [blk]
    args = [z]
    for w, b in zip(ws, bs):
        ins += [full, row]
        args += [w, b]
    return pl.pallas_call(
        _tc_post_chain_kernel,
        grid=(nb,),
        in_specs=ins,
        out_specs=[blk, pl.BlockSpec((8, FP), lambda i: (0, 0))],
        out_shape=[jax.ShapeDtypeStruct((N_NODES, FP), jnp.float32),
                   jax.ShapeDtypeStruct((8, FP), jnp.float32)],
        compiler_params=_VMEM_BIG,
    )(*args)


def _tc_head_kernel(z_ref, stat_ref, g_ref, be_ref, batch_ref,
                    w1_ref, b1_ref, w2_ref, b2_ref, w3_ref, b3_ref,
                    xf_ref, pool_ref):
    stat = stat_ref[...]
    mu = stat[0:1, :] * (1.0 / N_NODES)
    ex2 = stat[1:2, :] * (1.0 / N_NODES)
    var = ex2 - mu * mu
    rstd = jax.lax.rsqrt(var + 1e-5)
    z = (z_ref[...] - mu) * rstd * g_ref[...] + be_ref[...]
    h = jnp.maximum(z, 0.0)

    f = jnp.maximum(_mm(_split(h), (w1_ref[0], w1_ref[1])) + b1_ref[...],
                    0.0)
    f = jnp.maximum(_mm(_split(f), (w2_ref[0], w2_ref[1])) + b2_ref[...],
                    0.0)
    xf_ref[...] = _mm(_split(f), (w3_ref[0], w3_ref[1])) + b3_ref[...]

    onehot = (batch_ref[...] == lax.broadcasted_iota(
        jnp.int32, (1, N_GRAPHS), 1)).astype(jnp.float32)

    @pl.when(pl.program_id(0) == 0)
    def _():
        pool_ref[...] = jnp.zeros_like(pool_ref)
    pool_ref[...] += _dot3(onehot.T, h)


def _tc_head(z, stat, g, be, batch, w1, b1, w2, b2, w3, b3, bn=256):
    nb = N_NODES // bn
    blk = pl.BlockSpec((bn, FP), lambda i: (i, 0))
    return pl.pallas_call(
        _tc_head_kernel,
        grid=(nb,),
        in_specs=[blk,
                  pl.BlockSpec((8, FP), lambda i: (0, 0)),
                  pl.BlockSpec((1, FP), lambda i: (0, 0)),
                  pl.BlockSpec((1, FP), lambda i: (0, 0)),
                  pl.BlockSpec((bn, 1), lambda i: (i, 0)),
                  pl.BlockSpec((2, FP, 640), lambda i: (0, 0, 0)),
                  pl.BlockSpec((1, 640), lambda i: (0, 0)),
                  pl.BlockSpec((2, 640, 128), lambda i: (0, 0, 0)),
                  pl.BlockSpec((1, 128), lambda i: (0, 0)),
                  pl.BlockSpec((2, 128, 128), lambda i: (0, 0, 0)),
                  pl.BlockSpec((1, 128), lambda i: (0, 0))],
        out_specs=[pl.BlockSpec((bn, 128), lambda i: (i, 0)),
                   pl.BlockSpec((N_GRAPHS, FP), lambda i: (0, 0))],
        out_shape=[jax.ShapeDtypeStruct((N_NODES, 128), jnp.float32),
                   jax.ShapeDtypeStruct((N_GRAPHS, FP), jnp.float32)],
        compiler_params=_VMEM_BIG,
    )(z, stat, g, be, batch, w1, b1, w2, b2, w3, b3)


def _tc_energy_kernel(p_ref, w1_ref, b1_ref, w2_ref, b2_ref, w3_ref, b3_ref,
                      out_ref):
    f = jnp.maximum(_dot3(p_ref[...], w1_ref[...]) + b1_ref[...],
                    0.0)
    f = jnp.maximum(_dot3(f, w2_ref[...]) + b2_ref[...],
                    0.0)
    out_ref[...] = _dot3(f, w3_ref[...]) + b3_ref[...]


def _tc_energy(p, w1, b1, w2, b2, w3, b3):
    return pl.pallas_call(
        _tc_energy_kernel,
        out_shape=jax.ShapeDtypeStruct((N_GRAPHS, 128), jnp.float32),
    )(p, w1, b1, w2, b2, w3, b3)


def kernel(x, edge_index, edge_attr, batch, params):
    xp = jnp.pad(x, ((0, 0), (0, FP - F)))
    src = edge_index[0]
    dst = edge_index[1]

    w1 = _pad2(params["mlp1"]["w"], FP, FP)
    b1 = _pad1(params["mlp1"]["b"], FP).reshape(1, FP)

    pre0w = params["pre"][0]["w"]
    wd = _pad2(pre0w[:F], FP, FP)
    ws = _pad2(pre0w[F:2 * F], FP, FP)
    we = _pad2(pre0w[2 * F:], FP, FP)
    b0pre = _pad1(params["pre"][0]["b"], FP).reshape(1, FP)

    emb = _pad2(params["edge_emb"], 32, 16)
    encw = _pad2(params["edge_enc"]["w"], 16, FP)
    encb = _pad1(params["edge_enc"]["b"], FP).reshape(1, FP)

    prew = [_pad2(p["w"], FP, FP) for p in params["pre"][1:]]
    preb = [_pad1(p["b"], FP).reshape(1, FP) for p in params["pre"][1:]]

    aw = jax.nn.softmax(params["agg_w"])
    post0w = params["post"][0]["w"]
    wh = _pad2(post0w[:F], FP, FP)
    p1 = _pad2(post0w[F:2 * F], FP, FP) * aw[0]
    p2 = _pad2(post0w[2 * F:3 * F], FP, FP) * aw[1]
    p3 = _pad2(post0w[3 * F:4 * F], FP, FP) * aw[2]
    p4 = _pad2(post0w[4 * F:5 * F], FP, FP) * aw[3]
    p5 = _pad2(post0w[5 * F:], FP, FP) * aw[4]
    b0post = _pad1(params["post"][0]["b"], FP).reshape(1, FP)

    postw = [_pad2(p["w"], FP, FP) for p in params["post"][1:]]
    postb = [_pad1(p["b"], FP).reshape(1, FP) for p in params["post"][1:]]
    postw.append(_pad2(params["lin"]["w"], FP, FP))
    postb.append(_pad1(params["lin"]["b"], FP).reshape(1, FP))

    g = _pad1(params["bn_gamma"], FP).reshape(1, FP)
    be = _pad1(params["bn_beta"], FP).reshape(1, FP)

    m3 = params["mlp3"]
    m3w1 = _pad2(m3[0]["w"], FP, 640)
    m3b1 = _pad1(m3[0]["b"], 640).reshape(1, 640)
    m3w2 = _pad2(m3[1]["w"], 640, 128)
    m3b2 = _pad1(m3[1]["b"], 128).reshape(1, 128)
    m3w3 = _pad2(m3[2]["w"], 128, 128)
    m3b3 = _pad1(m3[2]["b"], 128).reshape(1, 128)

    m2 = params["mlp2"]
    m2w1 = _pad2(m2[0]["w"], FP, 640)
    m2b1 = _pad1(m2[0]["b"], 640).reshape(1, 640)
    m2w2 = _pad2(m2[1]["w"], 640, 128)
    m2b2 = _pad1(m2[1]["b"], 128).reshape(1, 128)
    m2w3 = _pad2(m2[2]["w"], 128, 128)
    m2b3 = _pad1(m2[2]["b"], 128).reshape(1, 128)

    w1s = _wsplit(w1)
    wds = _wsplit(wd)
    wss = _wsplit(ws)
    whs = _wsplit(wh)
    prews = [_wsplit(w) for w in prew]
    p1s, p2s, p3s, p4s, p5s = (_wsplit(p) for p in (p1, p2, p3, p4, p5))
    postws = [_wsplit(w) for w in postw]
    m3w1s, m3w2s, m3w3s = _wsplit(m3w1), _wsplit(m3w2), _wsplit(m3w3)

    a, b, c = _tc_node(xp, w1s, b1, wds, wss, whs)
    t = _tc_table(emb, encw, encb, we, b0pre)
    m0 = _sc_gather(a, b, dst, src)
    m = _tc_edge(m0, edge_attr.astype(jnp.int32).reshape(N_EDGES, 1), t,
                 prews, preb)
    s, q, mn, mx, cnt = _sc_agg(m, dst)
    z0 = _tc_post0(c, s, q, mn, mx, cnt, p1s, p2s, p3s, p4s, p5s, b0post)
    z, stat = _tc_post_chain(z0, postws, postb)
    xf, pool = _tc_head(z, stat, g, be, batch.astype(jnp.int32).reshape(
        N_NODES, 1), m3w1s, m3b1, m3w2s, m3b2, m3w3s, m3b3)
    xe = _tc_energy(pool, m2w1, m2b1, m2w2, m2b2, m2w3, m2b3)

    force = xf[:, :3]
    energy = xe[:, :1]
    return force, energy


# speedup vs baseline: 1.2752x; 1.2430x over previous
import functools

import jax
import jax.numpy as jnp
from jax import lax
from jax.experimental import pallas as pl
from jax.experimental.pallas import tpu as pltpu
from jax.experimental.pallas import tpu_sc as plsc

def _split(a):
    ah = a.astype(jnp.bfloat16)
    al = (a - ah.astype(jnp.float32)).astype(jnp.bfloat16)
    return ah, al


def _mm(a2, b2):
    ah, al = a2
    bh, bl = b2
    d = jnp.dot(al, bh, preferred_element_type=jnp.float32)
    d = d + jnp.dot(ah, bl, preferred_element_type=jnp.float32)
    d = d + jnp.dot(ah, bh, preferred_element_type=jnp.float32)
    return d


def _dot3(a, b):
    return _mm(_split(a), _split(b))


_VMEM_BIG = pltpu.CompilerParams(vmem_limit_bytes=100 * 1024 * 1024)

F = 1262
FP = 1280
N_NODES = 5120
N_EDGES = 15360
N_GRAPHS = 64

NC = 2
NS = 16
NW = NC * NS
L = 16


def _pad2(w, r, c):
    return jnp.pad(w, ((0, r - w.shape[0]), (0, c - w.shape[1])))


def _pad1(b, n):
    return jnp.pad(b, (0, n - b.shape[0]))


def _wsplit(w):
    hi = w.astype(jnp.bfloat16)
    lo = (w - hi.astype(jnp.float32)).astype(jnp.bfloat16)
    return jnp.stack([hi, lo])


def _tc_node_kernel(x_ref, w1_ref, b1_ref, wd_ref, ws_ref, wh_ref,
                    a_ref, b_ref, c_ref):
    h0 = jnp.maximum(
        _mm(_split(x_ref[...]), (w1_ref[0], w1_ref[1]))
        + b1_ref[...], 0.0)
    h2 = _split(h0)
    a_ref[...] = _mm(h2, (wd_ref[0], wd_ref[1]))
    b_ref[...] = _mm(h2, (ws_ref[0], ws_ref[1]))
    c_ref[...] = _mm(h2, (wh_ref[0], wh_ref[1]))


def _tc_node(x, w1, b1, wd, ws, wh, bn=512):
    nb = N_NODES // bn
    full = pl.BlockSpec((2, FP, FP), lambda i: (0, 0, 0))
    row = pl.BlockSpec((1, FP), lambda i: (0, 0))
    blk = pl.BlockSpec((bn, FP), lambda i: (i, 0))
    return pl.pallas_call(
        _tc_node_kernel,
        grid=(nb,),
        in_specs=[blk, full, row, full, full, full],
        out_specs=[blk, blk, blk],
        out_shape=[jax.ShapeDtypeStruct((N_NODES, FP), jnp.float32)] * 3,
        compiler_params=_VMEM_BIG,
    )(x, w1, b1, wd, ws, wh)


def _tc_table_kernel(emb_ref, encw_ref, encb_ref, we_ref, b0_ref, t_ref):
    e = _dot3(emb_ref[...], encw_ref[...]) + encb_ref[...]
    t_ref[...] = _dot3(e, we_ref[...]) + b0_ref[...]


def _tc_table(emb, encw, encb, we, b0):
    return pl.pallas_call(
        _tc_table_kernel,
        out_shape=jax.ShapeDtypeStruct((32, FP), jnp.float32),
    )(emb, encw, encb, we, b0)


_G_CH = 32


def _sc_gather_body(a_hbm, b_hbm, dst_hbm, src_hbm, out_hbm,
                    dbuf, sbuf, bufa, bufb, sem1, sem2):
    wid = lax.axis_index("s") * NC + lax.axis_index("c")
    epw = N_EDGES // NW
    base = wid * epw
    nch = epw // _G_CH

    def chunk(i, _):
        off = base + i * _G_CH
        pltpu.sync_copy(dst_hbm.at[pl.ds(off, _G_CH)], dbuf)
        pltpu.sync_copy(src_hbm.at[pl.ds(off, _G_CH)], sbuf)
        cpa = pltpu.async_copy(a_hbm.at[dbuf], bufa, sem1)
        cpb = pltpu.async_copy(b_hbm.at[sbuf], bufb, sem2)
        cpa.wait()
        cpb.wait()

        def row(j, _):
            for k in range(FP // L):
                sl = pl.ds(k * L, L)
                bufa[j, sl] = bufa[j, sl] + bufb[j, sl]
            return 0

        lax.fori_loop(0, _G_CH, row, 0)
        pltpu.sync_copy(bufa, out_hbm.at[pl.ds(off, _G_CH)])
        return 0

    lax.fori_loop(0, nch, chunk, 0)


def _sc_gather(a, b, dst, src):
    mesh = plsc.VectorSubcoreMesh(core_axis_name="c", subcore_axis_name="s",
                                  num_cores=NC, num_subcores=NS)
    fn = functools.partial(
        pl.kernel,
        out_type=jax.ShapeDtypeStruct((N_EDGES, FP), jnp.float32),
        mesh=mesh,
        compiler_params=pltpu.CompilerParams(needs_layout_passes=False),
        scratch_types=[
            pltpu.VMEM((_G_CH,), jnp.int32),
            pltpu.VMEM((_G_CH,), jnp.int32),
            pltpu.VMEM((_G_CH, FP), jnp.float32),
            pltpu.VMEM((_G_CH, FP), jnp.float32),
            pltpu.SemaphoreType.DMA,
            pltpu.SemaphoreType.DMA,
        ],
    )(_sc_gather_body)
    return fn(a, b, dst, src)


def _tc_edge_kernel(m0_ref, attr_ref, t_ref, w1_ref, b1_ref, w2_ref, b2_ref,
                    w3_ref, b3_ref, w4_ref, b4_ref, out_ref):
    attr = attr_ref[...]
    onehot = (attr == lax.broadcasted_iota(jnp.int32, (1, 32), 1)
              ).astype(jnp.float32)
    h = attr.shape[0] // 2
    zs = [m0_ref[0:h, :] + _dot3(onehot[0:h], t_ref[...]),
          m0_ref[h:, :] + _dot3(onehot[h:], t_ref[...])]
    for w_ref, b_ref in ((w1_ref, b1_ref), (w2_ref, b2_ref),
                         (w3_ref, b3_ref), (w4_ref, b4_ref)):
        zs = [_mm(_split(jnp.maximum(z, 0.0)),
                  (w_ref[0], w_ref[1])) + b_ref[...] for z in zs]
    out_ref[0:h, :] = zs[0]
    out_ref[h:, :] = zs[1]


def _tc_edge(m0, attr, t, ws, bs, bn=512):
    nb = N_EDGES // bn
    full = pl.BlockSpec((2, FP, FP), lambda i: (0, 0, 0))
    row = pl.BlockSpec((1, FP), lambda i: (0, 0))
    blk = pl.BlockSpec((bn, FP), lambda i: (i, 0))
    ins = [blk,
           pl.BlockSpec((bn, 1), lambda i: (i, 0)),
           pl.BlockSpec((32, FP), lambda i: (0, 0))]
    args = [m0, attr, t]
    for w, b in zip(ws, bs):
        ins += [full, row]
        args += [w, b]
    return pl.pallas_call(
        _tc_edge_kernel,
        grid=(nb,),
        in_specs=ins,
        out_specs=blk,
        out_shape=jax.ShapeDtypeStruct((N_EDGES, FP), jnp.float32),
        compiler_params=_VMEM_BIG,
    )(*args)


_R_NPW = N_NODES // NW
_R_FB = 128
_R_NFC = FP // _R_FB
_R_ELMAX = 2048
_R_CE = 128
_R_DCH = 1024

_NEG = -3.0e38
_POS = 3.0e38


def _sc_agg_body(m_hbm, dst_hbm, s_hbm, q_hbm, mn_hbm, mx_hbm, cnt_hbm,
                 dall, elist, dloc, rowid, rowbuf,
                 accs, accq, accmn, accmx, acccnt, sem):
    wid = lax.axis_index("s") * NC + lax.axis_index("c")
    lo = wid * _R_NPW

    def zinit(k, _):
        z = jnp.zeros((L,), jnp.int32)
        elist[pl.ds(k * L, L)] = z
        dloc[pl.ds(k * L, L)] = z
        return 0
    lax.fori_loop(0, _R_ELMAX // L, zinit, 0)

    def dchunk(h, cur0):
        pltpu.sync_copy(dst_hbm.at[pl.ds(h * _R_DCH, _R_DCH)], dall)

        def scan(k, cur):
            v = dall[pl.ds(k * L, L)]
            msk = (v >= lo) & (v < lo + _R_NPW)
            lane = lax.iota(jnp.int32, L)
            eid = h * _R_DCH + k * L + lane
            mi = msk.astype(jnp.int32)
            pos = cur + plsc.cumsum(mi) - mi
            idx = jnp.where(msk, pos, _R_ELMAX + lane)
            plsc.store_scatter(elist, [idx], eid)
            plsc.store_scatter(dloc, [idx], v - lo)
            return cur + jnp.sum(mi)
        return lax.fori_loop(0, _R_DCH // L, scan, cur0)
    nloc = lax.fori_loop(0, N_EDGES // _R_DCH, dchunk, 0)

    def cinit(j, _):
        acccnt[j, :] = jnp.zeros((L,), jnp.float32)
        return 0
    lax.fori_loop(0, _R_NPW, cinit, 0)

    def ccount(e, _):
        d = dloc[pl.ds(e, L)][0]
        acccnt[d, :] = acccnt[d, :] + 1.0
        return 0
    lax.fori_loop(0, nloc, ccount, 0)
    pltpu.sync_copy(acccnt, cnt_hbm.at[pl.ds(lo, _R_NPW)])

    nchunks = (nloc + _R_CE - 1) // _R_CE

    def fchunk(c, _):
        def ainit(j, _):
            for k in range(_R_FB // L):
                sl = pl.ds(k * L, L)
                accs[j, sl] = jnp.zeros((L,), jnp.float32)
                accq[j, sl] = jnp.zeros((L,), jnp.float32)
                accmn[j, sl] = jnp.full((L,), _POS, jnp.float32)
                accmx[j, sl] = jnp.full((L,), _NEG, jnp.float32)
            return 0
        lax.fori_loop(0, _R_NPW, ainit, 0)

        def echunk(ec, _):
            ebase = ec * _R_CE

            def ridx(k, _):
                e = elist[pl.ds(ebase + k * L, L)]
                rowid[pl.ds(k * L, L)] = e * _R_NFC + c
                return 0
            lax.fori_loop(0, _R_CE // L, ridx, 0)
            pltpu.async_copy(m_hbm.at[rowid], rowbuf, sem).wait()

            nhere = jnp.minimum(nloc - ebase, _R_CE)

            def edge(e, _):
                d = dloc[pl.ds(ebase + e, L)][0]
                nk = _R_FB // L
                sls = [pl.ds(k * L, L) for k in range(nk)]
                vs = [rowbuf[e, sl] for sl in sls]
                olds = [accs[d, sl] for sl in sls]
                oldq = [accq[d, sl] for sl in sls]
                oldmn = [accmn[d, sl] for sl in sls]
                oldmx = [accmx[d, sl] for sl in sls]
                news = [o + v for o, v in zip(olds, vs)]
                newq = [o + v * v for o, v in zip(oldq, vs)]
                newmn = [jnp.minimum(o, v) for o, v in zip(oldmn, vs)]
                newmx = [jnp.maximum(o, v) for o, v in zip(oldmx, vs)]
                for k, sl in enumerate(sls):
                    accs[d, sl] = news[k]
                    accq[d, sl] = newq[k]
                    accmn[d, sl] = newmn[k]
                    accmx[d, sl] = newmx[k]
                return 0
            lax.fori_loop(0, nhere, edge, 0)
            return 0
        lax.fori_loop(0, nchunks, echunk, 0)

        fsl = pl.ds(c * _R_FB, _R_FB)
        nsl = pl.ds(lo, _R_NPW)
        pltpu.sync_copy(accs, s_hbm.at[nsl, fsl])
        pltpu.sync_copy(accq, q_hbm.at[nsl, fsl])
        pltpu.sync_copy(accmn, mn_hbm.at[nsl, fsl])
        pltpu.sync_copy(accmx, mx_hbm.at[nsl, fsl])
        return 0

    lax.fori_loop(0, _R_NFC, fchunk, 0)


def _sc_agg(m, dst):
    mesh = plsc.VectorSubcoreMesh(core_axis_name="c", subcore_axis_name="s",
                                  num_cores=NC, num_subcores=NS)
    mview = m.reshape(N_EDGES * _R_NFC, _R_FB)
    nf = jax.ShapeDtypeStruct((N_NODES, FP), jnp.float32)
    fn = functools.partial(
        pl.kernel,
        out_type=[nf, nf, nf, nf,
                  jax.ShapeDtypeStruct((N_NODES, L), jnp.float32)],
        mesh=mesh,
        compiler_params=pltpu.CompilerParams(needs_layout_passes=False),
        scratch_types=[
            pltpu.VMEM((_R_DCH,), jnp.int32),
            pltpu.VMEM((_R_ELMAX + L,), jnp.int32),
            pltpu.VMEM((_R_ELMAX + L,), jnp.int32),
            pltpu.VMEM((_R_CE,), jnp.int32),
            pltpu.VMEM((_R_CE, _R_FB), jnp.float32),
            pltpu.VMEM((_R_NPW, _R_FB), jnp.float32),
            pltpu.VMEM((_R_NPW, _R_FB), jnp.float32),
            pltpu.VMEM((_R_NPW, _R_FB), jnp.float32),
            pltpu.VMEM((_R_NPW, _R_FB), jnp.float32),
            pltpu.VMEM((_R_NPW, L), jnp.float32),
            pltpu.SemaphoreType.DMA,
        ],
    )(_sc_agg_body)
    return fn(mview, dst)


def _tc_post0_kernel(c_ref, s_ref, q_ref, mn_ref, mx_ref, cnt_ref,
                     p1_ref, p2_ref, p3_ref, p4_ref, p5_ref, b0_ref, out_ref):
    cnt = cnt_ref[...][:, 0:1]
    inv = 1.0 / jnp.maximum(cnt, 1.0)
    has = (cnt > 0.0).astype(jnp.float32)
    s = s_ref[...]
    mean = s * inv
    msq = q_ref[...] * inv
    std = jnp.sqrt(jnp.maximum(msq - mean * mean, 0.0) + 1e-5)
    mn = mn_ref[...] * has
    mx = mx_ref[...] * has
    acc = c_ref[...] + b0_ref[...]
    acc += _mm(_split(s), (p1_ref[0], p1_ref[1]))
    acc += _mm(_split(mean), (p2_ref[0], p2_ref[1]))
    acc += _mm(_split(mn), (p3_ref[0], p3_ref[1]))
    acc += _mm(_split(mx), (p4_ref[0], p4_ref[1]))
    acc += _mm(_split(std), (p5_ref[0], p5_ref[1]))
    out_ref[...] = acc


def _tc_post0(c, s, q, mn, mx, cnt, p1, p2, p3, p4, p5, b0, bn=256):
    nb = N_NODES // bn
    full = pl.BlockSpec((2, FP, FP), lambda i: (0, 0, 0))
    row = pl.BlockSpec((1, FP), lambda i: (0, 0))
    blk = pl.BlockSpec((bn, FP), lambda i: (i, 0))
    cblk = pl.BlockSpec((bn, L), lambda i: (i, 0))
    return pl.pallas_call(
        _tc_post0_kernel,
        grid=(nb,),
        in_specs=[blk, blk, blk, blk, blk, cblk,
                  full, full, full, full, full, row],
        out_specs=blk,
        out_shape=jax.ShapeDtypeStruct((N_NODES, FP), jnp.float32),
        compiler_params=_VMEM_BIG,
    )(c, s, q, mn, mx, cnt, p1, p2, p3, p4, p5, b0)


def _tc_post_chain_kernel(z_ref, w1_ref, b1_ref, w2_ref, b2_ref, w3_ref,
                          b3_ref, w4_ref, b4_ref, wl_ref, bl_ref,
                          out_ref, stat_ref):
    bn2 = z_ref.shape[0] // 2
    zs = [z_ref[0:bn2, :], z_ref[bn2:, :]]
    for w_ref, b_ref in ((w1_ref, b1_ref), (w2_ref, b2_ref),
                         (w3_ref, b3_ref), (w4_ref, b4_ref)):
        zs = [_mm(_split(jnp.maximum(z, 0.0)),
                  (w_ref[0], w_ref[1])) + b_ref[...] for z in zs]
    zs = [_mm(_split(z), (wl_ref[0], wl_ref[1])) + bl_ref[...] for z in zs]
    out_ref[0:bn2, :] = zs[0]
    out_ref[bn2:, :] = zs[1]
    z = jnp.concatenate(zs, axis=0)

    @pl.when(pl.program_id(0) == 0)
    def _():
        stat_ref[...] = jnp.zeros_like(stat_ref)
    stat_ref[0:1, :] += jnp.sum(z, axis=0, keepdims=True)
    stat_ref[1:2, :] += jnp.sum(z * z, axis=0, keepdims=True)


def _tc_post_chain(z, ws, bs, bn=256):
    nb = N_NODES // bn
    full = pl.BlockSpec((2, FP, FP), lambda i: (0, 0, 0))
    row = pl.BlockSpec((1, FP), lambda i: (0, 0))
    blk = pl.BlockSpec((bn, FP), lambda i: (i, 0))
    ins = [blk]
    args = [z]
    for w, b in zip(ws, bs):
        ins += [full, row]
        args += [w, b]
    return pl.pallas_call(
        _tc_post_chain_kernel,
        grid=(nb,),
        in_specs=ins,
        out_specs=[blk, pl.BlockSpec((8, FP), lambda i: (0, 0))],
        out_shape=[jax.ShapeDtypeStruct((N_NODES, FP), jnp.float32),
                   jax.ShapeDtypeStruct((8, FP), jnp.float32)],
        compiler_params=_VMEM_BIG,
    )(*args)


def _tc_head_kernel(z_ref, stat_ref, g_ref, be_ref, batch_ref,
                    w1_ref, b1_ref, w2_ref, b2_ref, w3_ref, b3_ref,
                    xf_ref, pool_ref):
    stat = stat_ref[...]
    mu = stat[0:1, :] * (1.0 / N_NODES)
    ex2 = stat[1:2, :] * (1.0 / N_NODES)
    var = ex2 - mu * mu
    rstd = jax.lax.rsqrt(var + 1e-5)
    z = (z_ref[...] - mu) * rstd * g_ref[...] + be_ref[...]
    h = jnp.maximum(z, 0.0)

    f = jnp.maximum(_mm(_split(h), (w1_ref[0], w1_ref[1])) + b1_ref[...],
                    0.0)
    f = jnp.maximum(_mm(_split(f), (w2_ref[0], w2_ref[1])) + b2_ref[...],
                    0.0)
    xf_ref[...] = _mm(_split(f), (w3_ref[0], w3_ref[1])) + b3_ref[...]

    onehot = (batch_ref[...] == lax.broadcasted_iota(
        jnp.int32, (1, N_GRAPHS), 1)).astype(jnp.float32)

    @pl.when(pl.program_id(0) == 0)
    def _():
        pool_ref[...] = jnp.zeros_like(pool_ref)
    pool_ref[...] += _dot3(onehot.T, h)


def _tc_head(z, stat, g, be, batch, w1, b1, w2, b2, w3, b3, bn=256):
    nb = N_NODES // bn
    blk = pl.BlockSpec((bn, FP), lambda i: (i, 0))
    return pl.pallas_call(
        _tc_head_kernel,
        grid=(nb,),
        in_specs=[blk,
                  pl.BlockSpec((8, FP), lambda i: (0, 0)),
                  pl.BlockSpec((1, FP), lambda i: (0, 0)),
                  pl.BlockSpec((1, FP), lambda i: (0, 0)),
                  pl.BlockSpec((bn, 1), lambda i: (i, 0)),
                  pl.BlockSpec((2, FP, 640), lambda i: (0, 0, 0)),
                  pl.BlockSpec((1, 640), lambda i: (0, 0)),
                  pl.BlockSpec((2, 640, 128), lambda i: (0, 0, 0)),
                  pl.BlockSpec((1, 128), lambda i: (0, 0)),
                  pl.BlockSpec((2, 128, 128), lambda i: (0, 0, 0)),
                  pl.BlockSpec((1, 128), lambda i: (0, 0))],
        out_specs=[pl.BlockSpec((bn, 128), lambda i: (i, 0)),
                   pl.BlockSpec((N_GRAPHS, FP), lambda i: (0, 0))],
        out_shape=[jax.ShapeDtypeStruct((N_NODES, 128), jnp.float32),
                   jax.ShapeDtypeStruct((N_GRAPHS, FP), jnp.float32)],
        compiler_params=_VMEM_BIG,
    )(z, stat, g, be, batch, w1, b1, w2, b2, w3, b3)


def _tc_energy_kernel(p_ref, w1_ref, b1_ref, w2_ref, b2_ref, w3_ref, b3_ref,
                      out_ref):
    f = jnp.maximum(_dot3(p_ref[...], w1_ref[...]) + b1_ref[...],
                    0.0)
    f = jnp.maximum(_dot3(f, w2_ref[...]) + b2_ref[...],
                    0.0)
    out_ref[...] = _dot3(f, w3_ref[...]) + b3_ref[...]


def _tc_energy(p, w1, b1, w2, b2, w3, b3):
    return pl.pallas_call(
        _tc_energy_kernel,
        out_shape=jax.ShapeDtypeStruct((N_GRAPHS, 128), jnp.float32),
    )(p, w1, b1, w2, b2, w3, b3)


def kernel(x, edge_index, edge_attr, batch, params):
    xp = jnp.pad(x, ((0, 0), (0, FP - F)))
    src = edge_index[0]
    dst = edge_index[1]

    w1 = _pad2(params["mlp1"]["w"], FP, FP)
    b1 = _pad1(params["mlp1"]["b"], FP).reshape(1, FP)

    pre0w = params["pre"][0]["w"]
    wd = _pad2(pre0w[:F], FP, FP)
    ws = _pad2(pre0w[F:2 * F], FP, FP)
    we = _pad2(pre0w[2 * F:], FP, FP)
    b0pre = _pad1(params["pre"][0]["b"], FP).reshape(1, FP)

    emb = _pad2(params["edge_emb"], 32, 16)
    encw = _pad2(params["edge_enc"]["w"], 16, FP)
    encb = _pad1(params["edge_enc"]["b"], FP).reshape(1, FP)

    prew = [_pad2(p["w"], FP, FP) for p in params["pre"][1:]]
    preb = [_pad1(p["b"], FP).reshape(1, FP) for p in params["pre"][1:]]

    aw = jax.nn.softmax(params["agg_w"])
    post0w = params["post"][0]["w"]
    wh = _pad2(post0w[:F], FP, FP)
    p1 = _pad2(post0w[F:2 * F], FP, FP) * aw[0]
    p2 = _pad2(post0w[2 * F:3 * F], FP, FP) * aw[1]
    p3 = _pad2(post0w[3 * F:4 * F], FP, FP) * aw[2]
    p4 = _pad2(post0w[4 * F:5 * F], FP, FP) * aw[3]
    p5 = _pad2(post0w[5 * F:], FP, FP) * aw[4]
    b0post = _pad1(params["post"][0]["b"], FP).reshape(1, FP)

    postw = [_pad2(p["w"], FP, FP) for p in params["post"][1:]]
    postb = [_pad1(p["b"], FP).reshape(1, FP) for p in params["post"][1:]]
    postw.append(_pad2(params["lin"]["w"], FP, FP))
    postb.append(_pad1(params["lin"]["b"], FP).reshape(1, FP))

    g = _pad1(params["bn_gamma"], FP).reshape(1, FP)
    be = _pad1(params["bn_beta"], FP).reshape(1, FP)

    m3 = params["mlp3"]
    m3w1 = _pad2(m3[0]["w"], FP, 640)
    m3b1 = _pad1(m3[0]["b"], 640).reshape(1, 640)
    m3w2 = _pad2(m3[1]["w"], 640, 128)
    m3b2 = _pad1(m3[1]["b"], 128).reshape(1, 128)
    m3w3 = _pad2(m3[2]["w"], 128, 128)
    m3b3 = _pad1(m3[2]["b"], 128).reshape(1, 128)

    m2 = params["mlp2"]
    m2w1 = _pad2(m2[0]["w"], FP, 640)
    m2b1 = _pad1(m2[0]["b"], 640).reshape(1, 640)
    m2w2 = _pad2(m2[1]["w"], 640, 128)
    m2b2 = _pad1(m2[1]["b"], 128).reshape(1, 128)
    m2w3 = _pad2(m2[2]["w"], 128, 128)
    m2b3 = _pad1(m2[2]["b"], 128).reshape(1, 128)

    w1s = _wsplit(w1)
    wds = _wsplit(wd)
    wss = _wsplit(ws)
    whs = _wsplit(wh)
    prews = [_wsplit(w) for w in prew]
    p1s, p2s, p3s, p4s, p5s = (_wsplit(p) for p in (p1, p2, p3, p4, p5))
    postws = [_wsplit(w) for w in postw]
    m3w1s, m3w2s, m3w3s = _wsplit(m3w1), _wsplit(m3w2), _wsplit(m3w3)

    a, b, c = _tc_node(xp, w1s, b1, wds, wss, whs)
    t = _tc_table(emb, encw, encb, we, b0pre)
    m0 = _sc_gather(a, b, dst, src)
    m = _tc_edge(m0, edge_attr.astype(jnp.int32).reshape(N_EDGES, 1), t,
                 prews, preb)
    s, q, mn, mx, cnt = _sc_agg(m, dst)
    z0 = _tc_post0(c, s, q, mn, mx, cnt, p1s, p2s, p3s, p4s, p5s, b0post)
    z, stat = _tc_post_chain(z0, postws, postb)
    xf, pool = _tc_head(z, stat, g, be, batch.astype(jnp.int32).reshape(
        N_NODES, 1), m3w1s, m3b1, m3w2s, m3b2, m3w3s, m3b3)
    xe = _tc_energy(pool, m2w1, m2b1, m2w2, m2b2, m2w3, m2b3)

    force = xf[:, :3]
    energy = xe[:, :1]
    return force, energy
